# baseline (speedup 1.0000x reference)
import functools

import jax
import jax.numpy as jnp
from jax import lax
from jax.experimental import pallas as pl
from jax.experimental.pallas import tpu as pltpu
from jax.experimental.pallas import tpu_sc as plsc

_NC = 2
_NS = 16
_NW = _NC * _NS
_BM = 512


def _router_body(x_ref, c_ref, b_ref, idx_ref, w_ref):
    x = x_ref[...]
    c = c_ref[...]
    t, e = x.shape[0], c.shape[0]
    logits = lax.dot_general(x, c, (((1,), (1,)), ((), ())),
                             preferred_element_type=jnp.float32)
    a = jax.nn.sigmoid(logits) + b_ref[...]
    ii = lax.broadcasted_iota(jnp.int32, (t, e), 1)
    m1 = jnp.max(a, axis=1, keepdims=True)
    i1 = jnp.min(jnp.where(a == m1, ii, e), axis=1, keepdims=True)
    a2 = jnp.where(ii == i1, -jnp.inf, a)
    m2 = jnp.max(a2, axis=1, keepdims=True)
    i2 = jnp.min(jnp.where(a2 == m2, ii, e), axis=1, keepdims=True)
    em = jnp.exp(m2 - m1)
    denom = 1.0 + em
    idx_ref[...] = jnp.concatenate([i1, i2], axis=1)
    w_ref[...] = jnp.concatenate([1.0 / denom, em / denom], axis=1)


def _router(xf, centroids, routing_bias):
    t = xf.shape[0]
    return pl.pallas_call(
        _router_body,
        out_shape=[
            jax.ShapeDtypeStruct((t, 2), jnp.int32),
            jax.ShapeDtypeStruct((t, 2), jnp.float32),
        ],
    )(xf, centroids, routing_bias.reshape(1, -1))


def _dispatch_metadata(idx2, e, bm, nblk):
    flat_idx = idx2.reshape(-1)
    oh = (flat_idx[:, None] == jnp.arange(e, dtype=jnp.int32)[None, :])
    oh = oh.astype(jnp.int32)
    csum = jnp.cumsum(oh, axis=0)
    counts = csum[-1]
    rank = jnp.sum(csum * oh, axis=1) - 1
    pc = (counts + bm - 1) // bm
    ends = jnp.cumsum(pc)
    nblocks = ends[-1]
    start_pad = (ends - pc) * bm
    dest = start_pad[flat_idx] + rank
    d2 = dest.reshape(-1, 2)
    blk = jnp.arange(nblk, dtype=jnp.int32)
    be_raw = jnp.sum((ends[None, :] <= blk[:, None]).astype(jnp.int32), axis=1)
    eids = jnp.arange(e, dtype=jnp.int32)
    maxused = jnp.max(jnp.where(counts > 0, eids, 0))
    block_expert = jnp.minimum(be_raw, maxused)
    used = (counts > 0).astype(jnp.int32)
    pos_of_expert = jnp.cumsum(used) - 1
    n_pos = jnp.sum(used)
    ue = jnp.zeros((e,), jnp.int32).at[
        jnp.where(used > 0, pos_of_expert, e)].set(eids)
    ue = jnp.where(eids < n_pos, ue, maxused)
    bpos = pos_of_expert[block_expert]
    meta = jnp.concatenate(
        [bpos, ue, n_pos[None], nblocks[None]]).astype(jnp.int32)
    return d2[:, 0], d2[:, 1], meta, counts


def _dispatch(xf, dest_a, dest_b, pad_rows):
    t, h = xf.shape
    tpw = t // _NW
    mesh = plsc.VectorSubcoreMesh(core_axis_name="c", subcore_axis_name="s")

    @functools.partial(
        pl.kernel, mesh=mesh,
        out_type=jax.ShapeDtypeStruct((pad_rows, h), jnp.float32),
        scratch_types=[
            pltpu.VMEM((tpw, h), jnp.float32),
            pltpu.VMEM((tpw,), jnp.int32),
            pltpu.VMEM((tpw,), jnp.int32),
            pltpu.SemaphoreType.DMA,
            pltpu.SemaphoreType.DMA,
        ],
    )
    def k(xf_hbm, da_hbm, db_hbm, xp_hbm, xv, ia, ib, sa, sb):
        wid = lax.axis_index("s") * _NC + lax.axis_index("c")
        base = wid * tpw
        pltpu.sync_copy(xf_hbm.at[pl.ds(base, tpw)], xv)
        pltpu.sync_copy(da_hbm.at[pl.ds(base, tpw)], ia)
        pltpu.sync_copy(db_hbm.at[pl.ds(base, tpw)], ib)
        ca = pltpu.async_copy(xv, xp_hbm.at[ia], sa)
        cb = pltpu.async_copy(xv, xp_hbm.at[ib], sb)
        ca.wait()
        cb.wait()

    return k(xf, dest_a, dest_b)


def _ffn_body(nblk, s_ref, x_ref, wg_any, wu_any, wd_any, y_ref,
              wgb, wub, wdb, sems):
    b = pl.program_id(0)
    p = s_ref[b]
    n_pos = s_ref[nblk + 8]
    nblocks = s_ref[nblk + 9]
    prev_p = s_ref[jnp.maximum(b - 1, 0)]
    first = jnp.logical_or(b == 0, p != prev_p)

    def copies(q):
        eq = s_ref[nblk + q]
        slot = lax.rem(q, 3)
        out = []
        for src, dst in ((wg_any, wgb), (wu_any, wub), (wd_any, wdb)):
            hh = src.shape[1]
            for i in range(2):
                sl = pl.ds(i * (hh // 2), hh // 2)
                out.append(pltpu.make_async_copy(
                    src.at[eq, sl], dst.at[slot, sl], sems.at[slot]))
        return out

    @pl.when(b == 0)
    def _():
        for q in range(3):
            @pl.when(q < n_pos)
            def _():
                for c in copies(jnp.int32(q)):
                    c.start()

    @pl.when(jnp.logical_and(b > 0, first))
    def _():
        @pl.when(p + 2 < n_pos)
        def _():
            for c in copies(p + 2):
                c.start()

    @pl.when(first)
    def _():
        for c in copies(p):
            c.wait()

    @pl.when(b < nblocks)
    def _():
        slot = lax.rem(p, 3)
        xb = x_ref[...]
        g = jnp.dot(xb, wgb[slot], preferred_element_type=jnp.float32)
        u = jnp.dot(xb, wub[slot], preferred_element_type=jnp.float32)
        h = g * jax.nn.sigmoid(g) * u
        y_ref[...] = jnp.dot(h, wdb[slot], preferred_element_type=jnp.float32)


def _grouped_ffn(meta, xp, Wg, Wu, Wd, nblk):
    pad_rows, h = xp.shape
    dff = Wg.shape[2]
    grid_spec = pltpu.PrefetchScalarGridSpec(
        num_scalar_prefetch=1,
        grid=(nblk,),
        in_specs=[
            pl.BlockSpec((_BM, h), lambda b, s: (b, 0)),
            pl.BlockSpec(memory_space=pl.ANY),
            pl.BlockSpec(memory_space=pl.ANY),
            pl.BlockSpec(memory_space=pl.ANY),
        ],
        out_specs=pl.BlockSpec((_BM, h), lambda b, s: (b, 0)),
        scratch_shapes=[
            pltpu.VMEM((3, h, dff), jnp.float32),
            pltpu.VMEM((3, h, dff), jnp.float32),
            pltpu.VMEM((3, dff, h), jnp.float32),
            pltpu.SemaphoreType.DMA((3,)),
        ],
    )
    return pl.pallas_call(
        functools.partial(_ffn_body, nblk),
        grid_spec=grid_spec,
        out_shape=jax.ShapeDtypeStruct((pad_rows, h), jnp.float32),
    )(meta, xp, Wg, Wu, Wd)


def _combine(yp, dest_a, dest_b, w1b, w2b):
    t = dest_a.shape[0]
    h = yp.shape[1]
    tpw = t // _NW
    ch = 32
    mesh = plsc.VectorSubcoreMesh(core_axis_name="c", subcore_axis_name="s")

    @functools.partial(
        pl.kernel, mesh=mesh,
        out_type=jax.ShapeDtypeStruct((t, h), jnp.float32),
        scratch_types=[
            pltpu.VMEM((ch, h), jnp.float32),
            pltpu.VMEM((ch, h), jnp.float32),
            pltpu.VMEM((ch,), jnp.int32),
            pltpu.VMEM((ch,), jnp.int32),
            pltpu.VMEM((ch, 16), jnp.float32),
            pltpu.VMEM((ch, 16), jnp.float32),
            pltpu.SemaphoreType.DMA,
            pltpu.SemaphoreType.DMA,
        ],
    )
    def k(yp_hbm, da_hbm, db_hbm, w1_hbm, w2_hbm, out_hbm,
          av, bv, ia, ib, wa, wb, sa, sb):
        wid = lax.axis_index("s") * _NC + lax.axis_index("c")
        for j in range(tpw // ch):
            base = wid * tpw + j * ch
            pltpu.sync_copy(da_hbm.at[pl.ds(base, ch)], ia)
            pltpu.sync_copy(db_hbm.at[pl.ds(base, ch)], ib)
            pltpu.sync_copy(w1_hbm.at[pl.ds(base, ch)], wa)
            pltpu.sync_copy(w2_hbm.at[pl.ds(base, ch)], wb)
            pltpu.async_copy(yp_hbm.at[ia], av, sa).wait()
            pltpu.async_copy(yp_hbm.at[ib], bv, sb).wait()

            def body(tt, carry):
                wav = wa[tt, :]
                wbv = wb[tt, :]
                for cc in range(h // 16):
                    sl = pl.ds(cc * 16, 16)
                    av[tt, sl] = av[tt, sl] * wav + bv[tt, sl] * wbv
                return carry

            lax.fori_loop(0, ch, body, 0)
            pltpu.sync_copy(av, out_hbm.at[pl.ds(base, ch)])

    return k(yp, dest_a, dest_b, w1b, w2b)


def kernel(x, centroids, routing_bias, Wg, Wu, Wd):
    bb, ss, h = x.shape
    e = centroids.shape[0]
    t = bb * ss
    nblk = (2 * t) // _BM + e - 1
    pad_rows = nblk * _BM

    xf = x.reshape(t, h)
    idx2, w2k = _router(xf, centroids, routing_bias)
    dest_a, dest_b, meta, counts = _dispatch_metadata(idx2, e, _BM, nblk)
    xp = _dispatch(xf, dest_a, dest_b, pad_rows)
    yp = _grouped_ffn(meta, xp, Wg, Wu, Wd, nblk)
    w1b = jnp.broadcast_to(w2k[:, 0:1], (t, 16))
    w2b = jnp.broadcast_to(w2k[:, 1:2], (t, 16))
    out = _combine(yp, dest_a, dest_b, w1b, w2b)
    return out.reshape(bb, ss, h), counts


# speedup vs baseline: 1.1178x; 1.1178x over previous
import functools

import jax
import jax.numpy as jnp
from jax import lax
from jax.experimental import pallas as pl
from jax.experimental.pallas import tpu as pltpu
from jax.experimental.pallas import tpu_sc as plsc

_NC = 2
_NS = 16
_NW = _NC * _NS
_BM = 512


_UE_OFF = 16
_NPOS_OFF = 24
_NBLK_OFF = 25


def _router_body(nblk, x_ref, c_ref, b_ref,
                 da_ref, db_ref, w1_ref, w2_ref, meta_ref, counts_ref):
    x = x_ref[...]
    c = c_ref[...]
    t, e = x.shape[0], c.shape[0]
    bm = jnp.float32(_BM)
    logits = lax.dot_general(x, c, (((1,), (1,)), ((), ())),
                             preferred_element_type=jnp.float32)
    a = jax.nn.sigmoid(logits) + b_ref[...]
    ii = lax.broadcasted_iota(jnp.int32, (t, e), 1)
    m1 = jnp.max(a, axis=1, keepdims=True)
    i1 = jnp.min(jnp.where(a == m1, ii, e), axis=1, keepdims=True)
    a2 = jnp.where(ii == i1, -jnp.inf, a)
    m2 = jnp.max(a2, axis=1, keepdims=True)
    i2 = jnp.min(jnp.where(a2 == m2, ii, e), axis=1, keepdims=True)
    em = jnp.exp(m2 - m1)
    denom = 1.0 + em
    w1_ref[...] = jnp.broadcast_to(1.0 / denom, (t, 16))
    w2_ref[...] = jnp.broadcast_to(em / denom, (t, 16))

    oh1 = (jnp.broadcast_to(i1, (t, e)) == ii).astype(jnp.float32)
    oh2 = (jnp.broadcast_to(i2, (t, e)) == ii).astype(jnp.float32)
    cnt = oh1 + oh2
    ch = 128
    tri = (lax.broadcasted_iota(jnp.int32, (ch, ch), 1)
           < lax.broadcasted_iota(jnp.int32, (ch, ch), 0)).astype(jnp.float32)
    segs = []
    carry = jnp.zeros((1, e), jnp.float32)
    for g in range(t // ch):
        cg = cnt[g * ch:(g + 1) * ch]
        segs.append(jnp.dot(tri, cg, preferred_element_type=jnp.float32)
                    + carry)
        carry = carry + jnp.sum(cg, axis=0, keepdims=True)
    s_excl = jnp.concatenate(segs, axis=0)
    counts = carry
    rank1 = jnp.sum(oh1 * s_excl, axis=1, keepdims=True)
    rank2 = jnp.sum(oh2 * (s_excl + oh1), axis=1, keepdims=True)

    lo = lax.broadcasted_iota(jnp.int32, (e, e), 0)
    hi = lax.broadcasted_iota(jnp.int32, (e, e), 1)
    tri_incl = (lo <= hi).astype(jnp.float32)
    pc = jnp.floor((counts + (bm - 1.0)) * (1.0 / bm))
    ends = jnp.dot(pc, tri_incl, preferred_element_type=jnp.float32)
    start_pad = (ends - pc) * bm
    dest1 = jnp.sum(oh1 * start_pad, axis=1, keepdims=True) + rank1
    dest2 = jnp.sum(oh2 * start_pad, axis=1, keepdims=True) + rank2
    da_ref[...] = dest1.astype(jnp.int32)
    db_ref[...] = dest2.astype(jnp.int32)
    counts_ref[...] = counts.astype(jnp.int32)

    eids = lax.broadcasted_iota(jnp.int32, (1, e), 1).astype(jnp.float32)
    used = (counts > 0.0).astype(jnp.float32)
    n_pos = jnp.sum(used, axis=1, keepdims=True)
    maxused = jnp.max(used * eids, axis=1, keepdims=True)
    pos_of_expert = jnp.dot(used, tri_incl,
                            preferred_element_type=jnp.float32) - 1.0
    nblocks = jnp.max(ends, axis=1, keepdims=True)

    blkcol = lax.broadcasted_iota(jnp.int32, (16, e), 0).astype(jnp.float32)
    be_raw = jnp.sum((jnp.broadcast_to(ends, (16, e)) <= blkcol).astype(
        jnp.float32), axis=1, keepdims=True)
    block_expert = jnp.minimum(be_raw, jnp.broadcast_to(maxused, (16, 1)))
    be_oh = (jnp.broadcast_to(block_expert, (16, e))
             == lax.broadcasted_iota(jnp.int32, (16, e), 1).astype(jnp.float32)).astype(
        jnp.float32)
    bpos = jnp.sum(be_oh * pos_of_expert, axis=1, keepdims=True)

    prow = lax.broadcasted_iota(jnp.int32, (e, e), 0).astype(jnp.float32)
    peq = ((jnp.broadcast_to(pos_of_expert, (e, e)) == prow)
           * jnp.broadcast_to(used, (e, e)))
    ue_raw = jnp.sum(peq * jnp.broadcast_to(eids, (e, e)), axis=1,
                     keepdims=True)
    prow1 = lax.broadcasted_iota(jnp.int32, (e, 1), 0).astype(jnp.float32)
    ue = jnp.where(prow1 < jnp.broadcast_to(n_pos, (e, 1)), ue_raw,
                   jnp.broadcast_to(maxused, (e, 1)))
    meta = jnp.concatenate(
        [bpos, ue, n_pos, nblocks, jnp.zeros((32 - e - 18, 1), jnp.float32)],
        axis=0)
    meta_ref[...] = meta.astype(jnp.int32)


def _router(xf, centroids, routing_bias, nblk):
    t = xf.shape[0]
    e = centroids.shape[0]
    assert nblk <= _UE_OFF
    return pl.pallas_call(
        functools.partial(_router_body, nblk),
        out_shape=[
            jax.ShapeDtypeStruct((t, 1), jnp.int32),
            jax.ShapeDtypeStruct((t, 1), jnp.int32),
            jax.ShapeDtypeStruct((t, 16), jnp.float32),
            jax.ShapeDtypeStruct((t, 16), jnp.float32),
            jax.ShapeDtypeStruct((32, 1), jnp.int32),
            jax.ShapeDtypeStruct((1, e), jnp.int32),
        ],
    )(xf, centroids, routing_bias.reshape(1, -1))


def _dispatch(xf, dest_a, dest_b, pad_rows):
    t, h = xf.shape
    tpw = t // _NW
    mesh = plsc.VectorSubcoreMesh(core_axis_name="c", subcore_axis_name="s")

    @functools.partial(
        pl.kernel, mesh=mesh,
        out_type=jax.ShapeDtypeStruct((pad_rows, h), jnp.float32),
        scratch_types=[
            pltpu.VMEM((tpw, h), jnp.float32),
            pltpu.VMEM((tpw,), jnp.int32),
            pltpu.VMEM((tpw,), jnp.int32),
            pltpu.SemaphoreType.DMA,
            pltpu.SemaphoreType.DMA,
        ],
    )
    def k(xf_hbm, da_hbm, db_hbm, xp_hbm, xv, ia, ib, sa, sb):
        wid = lax.axis_index("s") * _NC + lax.axis_index("c")
        base = wid * tpw
        pltpu.sync_copy(xf_hbm.at[pl.ds(base, tpw)], xv)
        pltpu.sync_copy(da_hbm.at[pl.ds(base, tpw)], ia)
        pltpu.sync_copy(db_hbm.at[pl.ds(base, tpw)], ib)
        ca = pltpu.async_copy(xv, xp_hbm.at[ia], sa)
        cb = pltpu.async_copy(xv, xp_hbm.at[ib], sb)
        ca.wait()
        cb.wait()

    return k(xf, dest_a, dest_b)


def _ffn_body(nblk, s_ref, x_ref, wg_any, wu_any, wd_any, y_ref,
              wgb, wub, wdb, sems):
    b = pl.program_id(0)
    p = s_ref[b]
    n_pos = s_ref[_NPOS_OFF]
    nblocks = s_ref[_NBLK_OFF]
    prev_p = s_ref[jnp.maximum(b - 1, 0)]
    first = jnp.logical_or(b == 0, p != prev_p)

    def copies(q):
        eq = s_ref[_UE_OFF + q]
        slot = lax.rem(q, 3)
        out = []
        for src, dst in ((wg_any, wgb), (wu_any, wub), (wd_any, wdb)):
            hh = src.shape[1]
            for i in range(2):
                sl = pl.ds(i * (hh // 2), hh // 2)
                out.append(pltpu.make_async_copy(
                    src.at[eq, sl], dst.at[slot, sl], sems.at[slot]))
        return out

    @pl.when(b == 0)
    def _():
        for q in range(3):
            @pl.when(q < n_pos)
            def _():
                for c in copies(jnp.int32(q)):
                    c.start()

    @pl.when(jnp.logical_and(b > 0, first))
    def _():
        @pl.when(p + 2 < n_pos)
        def _():
            for c in copies(p + 2):
                c.start()

    @pl.when(first)
    def _():
        for c in copies(p):
            c.wait()

    @pl.when(b < nblocks)
    def _():
        slot = lax.rem(p, 3)
        xb = x_ref[...]
        g = jnp.dot(xb, wgb[slot], preferred_element_type=jnp.float32)
        u = jnp.dot(xb, wub[slot], preferred_element_type=jnp.float32)
        h = g * jax.nn.sigmoid(g) * u
        y_ref[...] = jnp.dot(h, wdb[slot], preferred_element_type=jnp.float32)


def _grouped_ffn(meta, xp, Wg, Wu, Wd, nblk):
    pad_rows, h = xp.shape
    dff = Wg.shape[2]
    grid_spec = pltpu.PrefetchScalarGridSpec(
        num_scalar_prefetch=1,
        grid=(nblk,),
        in_specs=[
            pl.BlockSpec((_BM, h), lambda b, s: (b, 0)),
            pl.BlockSpec(memory_space=pl.ANY),
            pl.BlockSpec(memory_space=pl.ANY),
            pl.BlockSpec(memory_space=pl.ANY),
        ],
        out_specs=pl.BlockSpec((_BM, h), lambda b, s: (b, 0)),
        scratch_shapes=[
            pltpu.VMEM((3, h, dff), jnp.float32),
            pltpu.VMEM((3, h, dff), jnp.float32),
            pltpu.VMEM((3, dff, h), jnp.float32),
            pltpu.SemaphoreType.DMA((3,)),
        ],
    )
    return pl.pallas_call(
        functools.partial(_ffn_body, nblk),
        grid_spec=grid_spec,
        out_shape=jax.ShapeDtypeStruct((pad_rows, h), jnp.float32),
    )(meta, xp, Wg, Wu, Wd)


def _combine(yp, dest_a, dest_b, w1b, w2b):
    t = dest_a.shape[0]
    h = yp.shape[1]
    tpw = t // _NW
    ch = 32
    mesh = plsc.VectorSubcoreMesh(core_axis_name="c", subcore_axis_name="s")

    @functools.partial(
        pl.kernel, mesh=mesh,
        out_type=jax.ShapeDtypeStruct((t, h), jnp.float32),
        scratch_types=[
            pltpu.VMEM((ch, h), jnp.float32),
            pltpu.VMEM((ch, h), jnp.float32),
            pltpu.VMEM((ch,), jnp.int32),
            pltpu.VMEM((ch,), jnp.int32),
            pltpu.VMEM((ch, 16), jnp.float32),
            pltpu.VMEM((ch, 16), jnp.float32),
            pltpu.SemaphoreType.DMA,
            pltpu.SemaphoreType.DMA,
        ],
    )
    def k(yp_hbm, da_hbm, db_hbm, w1_hbm, w2_hbm, out_hbm,
          av, bv, ia, ib, wa, wb, sa, sb):
        wid = lax.axis_index("s") * _NC + lax.axis_index("c")
        for j in range(tpw // ch):
            base = wid * tpw + j * ch
            pltpu.sync_copy(da_hbm.at[pl.ds(base, ch)], ia)
            pltpu.sync_copy(db_hbm.at[pl.ds(base, ch)], ib)
            pltpu.sync_copy(w1_hbm.at[pl.ds(base, ch)], wa)
            pltpu.sync_copy(w2_hbm.at[pl.ds(base, ch)], wb)
            pltpu.async_copy(yp_hbm.at[ia], av, sa).wait()
            pltpu.async_copy(yp_hbm.at[ib], bv, sb).wait()

            def body(tt, carry):
                wav = wa[tt, :]
                wbv = wb[tt, :]
                for cc in range(h // 16):
                    sl = pl.ds(cc * 16, 16)
                    av[tt, sl] = av[tt, sl] * wav + bv[tt, sl] * wbv
                return carry

            lax.fori_loop(0, ch, body, 0)
            pltpu.sync_copy(av, out_hbm.at[pl.ds(base, ch)])

    return k(yp, dest_a, dest_b, w1b, w2b)


def kernel(x, centroids, routing_bias, Wg, Wu, Wd):
    bb, ss, h = x.shape
    e = centroids.shape[0]
    t = bb * ss
    nblk = (2 * t) // _BM + e - 1
    pad_rows = nblk * _BM

    xf = x.reshape(t, h)
    da2, db2, w1b, w2b, meta2, counts2 = _router(
        xf, centroids, routing_bias, nblk)
    dest_a = da2.reshape(t)
    dest_b = db2.reshape(t)
    meta = meta2.reshape(32)
    counts = counts2.reshape(e)
    xp = _dispatch(xf, dest_a, dest_b, pad_rows)
    yp = _grouped_ffn(meta, xp, Wg, Wu, Wd, nblk)
    out = _combine(yp, dest_a, dest_b, w1b, w2b)
    return out.reshape(bb, ss, h), counts
